# baseline (speedup 1.0000x reference)
import functools

import jax
import jax.numpy as jnp
from jax.experimental import pallas as pl
from jax.experimental.pallas import tpu as pltpu
from jax.experimental.pallas import tpu_sc as plsc

_R = 64
_K = 8
_SCALING = 16.0 / 64.0
_BLK = 512
_SC_BLK = 32


def _tc1_body(x_ref, at_ref, rwt_ref, h16_ref, lg_ref):
    x = x_ref[...].astype(jnp.bfloat16)
    h = jnp.dot(x, at_ref[...], preferred_element_type=jnp.float32)
    h16 = h.astype(jnp.bfloat16)
    lg_ref[...] = jnp.dot(h16, rwt_ref[...], preferred_element_type=jnp.float32)
    h16_ref[...] = h16


def _tc2_body(h16_ref, mask_ref, bt_ref, o_ref):
    hs = h16_ref[...] * mask_ref[...].astype(jnp.bfloat16)
    o_ref[...] = jnp.dot(hs, bt_ref[...], preferred_element_type=jnp.float32)


def _sc_mask(lg):
    n = lg.shape[0]
    mesh = plsc.VectorSubcoreMesh(core_axis_name="core",
                                  subcore_axis_name="subcore")

    @pl.kernel(out_type=jax.ShapeDtypeStruct((n, _R), jnp.float32), mesh=mesh,
               compiler_params=pltpu.CompilerParams(needs_layout_passes=False))
    def _k(lg_hbm, mask_hbm):
        def body(lg_vmem, mask_vmem):
            lanes = jax.lax.iota(jnp.int32, 16)
            first8 = lanes < 8

            def sort_desc(v):
                s, _ = plsc.sort_key_val(v, lanes, descending=True)
                return s

            def top16(a, b):
                return jnp.where(first8, a, jax.lax.rev(b, (0,)))

            @pl.loop(0, _SC_BLK)
            def _(t):
                v = [lg_vmem.at[t, pl.ds(c * 16, 16)][...] for c in range(4)]
                s = [sort_desc(v[c]) for c in range(4)]
                m01 = sort_desc(top16(s[0], s[1]))
                m23 = sort_desc(top16(s[2], s[3]))
                fin = sort_desc(top16(m01, m23))
                t8 = fin[7]
                count_gt = jnp.sum((fin > t8).astype(jnp.int32))
                need_eq = 8 - count_gt
                carry = jnp.int32(0)
                for c in range(4):
                    gt = v[c] > t8
                    eq = v[c] == t8
                    cum = jnp.cumsum(eq.astype(jnp.int32))
                    sel = eq & ((cum + carry) <= need_eq)
                    mask_c = jnp.where(gt | sel, 1.0, 0.0)
                    mask_vmem.at[t, pl.ds(c * 16, 16)][...] = mask_c
                    carry = carry + cum[15]

        pltpu.emit_pipeline(
            body,
            grid=(n // _SC_BLK,),
            in_specs=[pl.BlockSpec((_SC_BLK, _R), lambda i: (i, 0))],
            out_specs=[pl.BlockSpec((_SC_BLK, _R), lambda i: (i, 0))],
            core_axis_name=("core", "subcore"),
            dimension_semantics=(pltpu.PARALLEL,),
        )(lg_hbm, mask_hbm)

    return _k(lg)


@functools.partial(jax.jit, static_argnames=())
def kernel(x, A, B, Rw):
    bsz, seq, d = x.shape
    n = bsz * seq
    x2 = x.reshape(n, d)
    at = A.T.astype(jnp.bfloat16)
    bt = (B.T * _SCALING).astype(jnp.bfloat16)
    rwt = Rw.T.astype(jnp.bfloat16)
    nb = n // _BLK

    h16, lg = pl.pallas_call(
        _tc1_body,
        grid=(nb,),
        in_specs=[
            pl.BlockSpec((_BLK, d), lambda i: (i, 0)),
            pl.BlockSpec((d, _R), lambda i: (0, 0)),
            pl.BlockSpec((_R, _R), lambda i: (0, 0)),
        ],
        out_specs=[
            pl.BlockSpec((_BLK, _R), lambda i: (i, 0)),
            pl.BlockSpec((_BLK, _R), lambda i: (i, 0)),
        ],
        out_shape=[
            jax.ShapeDtypeStruct((n, _R), jnp.bfloat16),
            jax.ShapeDtypeStruct((n, _R), jnp.float32),
        ],
        compiler_params=pltpu.CompilerParams(
            dimension_semantics=("arbitrary",)),
    )(x2, at, rwt)

    mask = _sc_mask(lg)

    out = pl.pallas_call(
        _tc2_body,
        grid=(nb,),
        in_specs=[
            pl.BlockSpec((_BLK, _R), lambda i: (i, 0)),
            pl.BlockSpec((_BLK, _R), lambda i: (i, 0)),
            pl.BlockSpec((_R, d), lambda i: (0, 0)),
        ],
        out_specs=pl.BlockSpec((_BLK, d), lambda i: (i, 0)),
        out_shape=jax.ShapeDtypeStruct((n, d), jnp.float32),
        compiler_params=pltpu.CompilerParams(
            dimension_semantics=("arbitrary",)),
    )(h16, mask, bt)
    return out.reshape(bsz, seq, d)


# speedup vs baseline: 1.2833x; 1.2833x over previous
import functools

import jax
import jax.numpy as jnp
from jax.experimental import pallas as pl
from jax.experimental.pallas import tpu as pltpu

_R = 64
_K = 8
_SCALING = 16.0 / 64.0
_BLK = 512
_CHUNKS = 4


def _topk_keep(lg):
    lanes = jax.lax.broadcasted_iota(jnp.int32, lg.shape, 1).astype(jnp.float32)
    cur = lg
    keep = jnp.zeros(lg.shape, jnp.bool_)
    for _ in range(_K):
        m = jnp.max(cur, axis=1, keepdims=True)
        cand = cur == m
        sel_idx = jnp.min(jnp.where(cand, lanes, float(_R)), axis=1,
                          keepdims=True)
        sel = lanes == sel_idx
        keep = keep | sel
        cur = jnp.where(sel, -jnp.inf, cur)
    return keep


def _body(x_ref, at_ref, bt_ref, rwt_ref, o_ref, h16_ref, lg_ref):
    i = pl.program_id(0)
    nb = pl.num_programs(0) - 1

    @pl.when(i > 0)
    def _finish_prev():
        h16 = h16_ref[...]
        keep = _topk_keep(lg_ref[...])
        hs = jnp.where(keep, h16, jnp.bfloat16(0.0))
        o_ref[...] = jnp.dot(hs, bt_ref[...],
                             preferred_element_type=jnp.float32)

    @pl.when(i < nb)
    def _start_cur():
        x = x_ref[...].astype(jnp.bfloat16)
        h = jnp.dot(x, at_ref[...], preferred_element_type=jnp.float32)
        h16 = h.astype(jnp.bfloat16)
        lg_ref[...] = jnp.dot(h16, rwt_ref[...],
                              preferred_element_type=jnp.float32)
        h16_ref[...] = h16


@functools.partial(jax.jit, static_argnames=())
def kernel(x, A, B, Rw):
    bsz, seq, d = x.shape
    n = bsz * seq
    x2 = x.reshape(n, d)
    at = A.T.astype(jnp.bfloat16)
    bt = (B.T * _SCALING).astype(jnp.bfloat16)
    rwt = Rw.T.astype(jnp.bfloat16)
    nb = n // _BLK
    out = pl.pallas_call(
        _body,
        grid=(nb + 1,),
        in_specs=[
            pl.BlockSpec((_BLK, d), lambda i: (jnp.minimum(i, nb - 1), 0)),
            pl.BlockSpec((d, _R), lambda i: (0, 0)),
            pl.BlockSpec((_R, d), lambda i: (0, 0)),
            pl.BlockSpec((_R, _R), lambda i: (0, 0)),
        ],
        out_specs=pl.BlockSpec((_BLK, d), lambda i: (jnp.maximum(i - 1, 0), 0)),
        out_shape=jax.ShapeDtypeStruct((n, d), jnp.float32),
        scratch_shapes=[
            pltpu.VMEM((_BLK, _R), jnp.bfloat16),
            pltpu.VMEM((_BLK, _R), jnp.float32),
        ],
        compiler_params=pltpu.CompilerParams(
            dimension_semantics=("arbitrary",)),
    )(x2, at, bt, rwt)
    return out.reshape(bsz, seq, d)


# speedup vs baseline: 1.2838x; 1.0005x over previous
import functools

import jax
import jax.numpy as jnp
from jax.experimental import pallas as pl
from jax.experimental.pallas import tpu as pltpu

_R = 64
_K = 8
_SCALING = 16.0 / 64.0
_BLK = 512


def _topk_keep(lg):
    lanes = jax.lax.broadcasted_iota(jnp.int32, lg.shape, 1).astype(jnp.float32)
    cur = lg
    keep = jnp.zeros(lg.shape, jnp.bool_)
    for _ in range(_K):
        m = jnp.max(cur, axis=1, keepdims=True)
        cand = cur == m
        sel_idx = jnp.min(jnp.where(cand, lanes, float(_R)), axis=1,
                          keepdims=True)
        sel = lanes == sel_idx
        keep = keep | sel
        cur = jnp.where(sel, -jnp.inf, cur)
    return keep


def _body(x_ref, at_ref, bt_ref, rwt_ref, o_ref, h16_ref, lg_ref):
    i = pl.program_id(0)
    nb = pl.num_programs(0) - 1

    @pl.when(i > 0)
    def _finish_prev():
        h16 = h16_ref[...]
        keep = _topk_keep(lg_ref[...])
        hs = jnp.where(keep, h16, jnp.bfloat16(0.0))
        o_ref[...] = jnp.dot(hs, bt_ref[...],
                             preferred_element_type=jnp.float32)

    @pl.when(i < nb)
    def _start_cur():
        x = x_ref[...].astype(jnp.bfloat16)
        h = jnp.dot(x, at_ref[...], preferred_element_type=jnp.float32)
        h16 = h.astype(jnp.bfloat16)
        lg_ref[...] = jnp.dot(h16, rwt_ref[...],
                              preferred_element_type=jnp.float32)
        h16_ref[...] = h16


@functools.partial(jax.jit, static_argnames=())
def kernel(x, A, B, Rw):
    bsz, seq, d = x.shape
    n = bsz * seq
    x2 = x.reshape(n, d)
    at = A.T.astype(jnp.bfloat16)
    bt = (B.T * _SCALING).astype(jnp.bfloat16)
    rwt = Rw.T.astype(jnp.bfloat16)
    nb = n // _BLK
    out = pl.pallas_call(
        _body,
        grid=(nb + 1,),
        in_specs=[
            pl.BlockSpec((_BLK, d), lambda i: (jnp.minimum(i, nb - 1), 0)),
            pl.BlockSpec((d, _R), lambda i: (0, 0)),
            pl.BlockSpec((_R, d), lambda i: (0, 0)),
            pl.BlockSpec((_R, _R), lambda i: (0, 0)),
        ],
        out_specs=pl.BlockSpec((_BLK, d), lambda i: (jnp.maximum(i - 1, 0), 0)),
        out_shape=jax.ShapeDtypeStruct((n, d), jnp.float32),
        scratch_shapes=[
            pltpu.VMEM((_BLK, _R), jnp.bfloat16),
            pltpu.VMEM((_BLK, _R), jnp.float32),
        ],
        compiler_params=pltpu.CompilerParams(
            dimension_semantics=("arbitrary",)),
    )(x2, at, bt, rwt)
    return out.reshape(bsz, seq, d)
